# baseline (speedup 1.0000x reference)
import jax
import jax.numpy as jnp
from jax.experimental import pallas as pl


def _loss_kernel(e_ref, lab_ref, out_ref):
    e = e_ref[:].astype(jnp.float32)
    norm = jnp.sqrt(jnp.sum(e * e, axis=1, keepdims=True))
    en = (e * (1.0 / jnp.maximum(norm, 1e-12))).astype(jnp.bfloat16)
    s = 20.0 * jnp.dot(en, en.T, preferred_element_type=jnp.float32)

    lab = lab_ref[0, :]
    eqf = (lab[:, None] == lab[None, :]).astype(jnp.float32)

    ex = jnp.exp(s - 20.0)
    z = jnp.sum((1.0 - eqf) * ex, axis=1, keepdims=True)
    nll = jnp.log(ex + z) - (s - 20.0)

    total = jnp.sum(eqf * nll)
    count = jnp.sum(eqf)
    out_ref[:, :] = jnp.broadcast_to(total / count, (1, 1))


def kernel(embeddings, labels):
    n = embeddings.shape[0]
    lab2d = labels.astype(jnp.int32).reshape(1, n)
    out = pl.pallas_call(
        _loss_kernel,
        out_shape=jax.ShapeDtypeStruct((1, 1), jnp.float32),
    )(embeddings.astype(jnp.bfloat16), lab2d)
    return out[0, 0]


# speedup vs baseline: 1.4558x; 1.4558x over previous
import jax
import jax.numpy as jnp
from jax.experimental import pallas as pl


def _loss_kernel(e_ref, lab_ref, out_ref):
    e = e_ref[:]
    norm = jnp.sqrt(jnp.sum(e * e, axis=1, keepdims=True))
    en = (e * (1.0 / jnp.maximum(norm, 1e-12))).astype(jnp.bfloat16)
    s = 20.0 * jnp.dot(en, en.T, preferred_element_type=jnp.float32)

    lab = lab_ref[0, :]
    eqf = (lab[:, None] == lab[None, :]).astype(jnp.float32)

    ex = jnp.exp(s - 20.0)
    z = jnp.sum((1.0 - eqf) * ex, axis=1, keepdims=True)
    nll = jnp.log(ex + z) - (s - 20.0)

    total = jnp.sum(eqf * nll)
    count = jnp.sum(eqf)
    out_ref[:, :] = jnp.broadcast_to(total / count, (1, 1))


def kernel(embeddings, labels):
    n = embeddings.shape[0]
    lab2d = labels.astype(jnp.int32).reshape(1, n)
    out = pl.pallas_call(
        _loss_kernel,
        out_shape=jax.ShapeDtypeStruct((1, 1), jnp.float32),
    )(embeddings, lab2d)
    return out[0, 0]
